# baseline (speedup 1.0000x reference)
import jax
import jax.numpy as jnp
from jax.experimental import pallas as pl

_N = 100000
_D = 128
_BLOCK = 4000


def _sigmoid_block(p_ref, z_ref):
    z_ref[...] = jax.nn.sigmoid(p_ref[...])


def kernel(P, test):
    return pl.pallas_call(
        _sigmoid_block,
        grid=(_N // _BLOCK,),
        in_specs=[pl.BlockSpec((_BLOCK, _D), lambda i: (i, 0))],
        out_specs=pl.BlockSpec((_BLOCK, _D), lambda i: (i, 0)),
        out_shape=jax.ShapeDtypeStruct((_N, _D), jnp.float32),
    )(P)


# speedup vs baseline: 1.1461x; 1.1461x over previous
import jax
import jax.numpy as jnp
from jax.experimental import pallas as pl

_N = 100000
_D = 128
_BLOCK = 10000


def _sigmoid_block(p_ref, z_ref):
    z_ref[...] = jax.nn.sigmoid(p_ref[...])


def kernel(P, test):
    return pl.pallas_call(
        _sigmoid_block,
        grid=(_N // _BLOCK,),
        in_specs=[pl.BlockSpec((_BLOCK, _D), lambda i: (i, 0))],
        out_specs=pl.BlockSpec((_BLOCK, _D), lambda i: (i, 0)),
        out_shape=jax.ShapeDtypeStruct((_N, _D), jnp.float32),
    )(P)


# speedup vs baseline: 1.1709x; 1.0217x over previous
import jax
import jax.numpy as jnp
from jax.experimental import pallas as pl

_N = 100000
_D = 128
_BLOCK = 20000


def _sigmoid_block(p_ref, z_ref):
    z_ref[...] = jax.nn.sigmoid(p_ref[...])


def kernel(P, test):
    return pl.pallas_call(
        _sigmoid_block,
        grid=(_N // _BLOCK,),
        in_specs=[pl.BlockSpec((_BLOCK, _D), lambda i: (i, 0))],
        out_specs=pl.BlockSpec((_BLOCK, _D), lambda i: (i, 0)),
        out_shape=jax.ShapeDtypeStruct((_N, _D), jnp.float32),
    )(P)


# speedup vs baseline: 1.1766x; 1.0049x over previous
import jax
import jax.numpy as jnp
from jax.experimental import pallas as pl

_N = 100000
_D = 128
_BLOCK = 25000


def _sigmoid_block(p_ref, z_ref):
    z_ref[...] = jax.nn.sigmoid(p_ref[...])


def kernel(P, test):
    return pl.pallas_call(
        _sigmoid_block,
        grid=(_N // _BLOCK,),
        in_specs=[pl.BlockSpec((_BLOCK, _D), lambda i: (i, 0))],
        out_specs=pl.BlockSpec((_BLOCK, _D), lambda i: (i, 0)),
        out_shape=jax.ShapeDtypeStruct((_N, _D), jnp.float32),
    )(P)
